# baseline (speedup 1.0000x reference)
import functools

import jax
import jax.numpy as jnp
from jax import lax
from jax.experimental import pallas as pl
from jax.experimental.pallas import tpu as pltpu
from jax.experimental.pallas import tpu_sc as plsc


def _idx_body(x_ref, cb_ref, idx_ref, cbp_ref):
    M, D = x_ref.shape
    K = cb_ref.shape[0]
    Dp = cbp_ref.shape[1]
    xv = x_ref[...]
    cb = cb_ref[...]
    x2 = jnp.sum(xv * xv, axis=1, keepdims=True)
    c2 = jnp.sum(cb * cb, axis=1)[None, :]
    xc = jax.lax.dot_general(xv, cb, (((1,), (1,)), ((), ())),
                             preferred_element_type=jnp.float32)
    dist = x2 + c2 - 2.0 * xc
    mins = jnp.min(dist, axis=1, keepdims=True)
    kio = jax.lax.broadcasted_iota(jnp.int32, (M, K), 1)
    idx_ref[...] = jnp.min(jnp.where(dist == mins, kio, K), axis=1)
    cbp_ref[...] = jnp.concatenate(
        [cb, jnp.zeros((K, Dp - D), jnp.float32)], axis=1)


def _make_sc_gather(K, D, Dp, B):
    info = plsc.get_sparse_core_info()
    nw = info.num_cores * info.num_subcores
    b_per_w = B // nw
    mesh = plsc.VectorSubcoreMesh(core_axis_name="c", subcore_axis_name="s")

    @functools.partial(
        pl.kernel, mesh=mesh,
        compiler_params=pltpu.CompilerParams(use_tc_tiling_on_sc=False),
        out_type=jax.ShapeDtypeStruct((B, Dp), jnp.float32),
        scratch_types=[
            pltpu.VMEM((b_per_w,), jnp.int32),
            pltpu.VMEM((b_per_w, Dp), jnp.float32),
            pltpu.SemaphoreType.DMA,
        ],
    )
    def _gather(idx_hbm, table_hbm, out_hbm, idx_v, rows_v, sem):
        wid = lax.axis_index("s") * info.num_cores + lax.axis_index("c")
        base = wid * b_per_w
        pltpu.sync_copy(idx_hbm.at[pl.ds(base, b_per_w)], idx_v)
        pltpu.async_copy(table_hbm.at[idx_v], rows_v, sem).wait()
        pltpu.sync_copy(rows_v, out_hbm.at[pl.ds(base, b_per_w)])

    return _gather


def kernel(x, codebook):
    N, C, H, W = x.shape
    K = codebook.shape[0]
    D = H * W
    M = N * C
    x_flat = x.reshape(M, D)
    cb_flat = codebook.reshape(K, D)
    Dp = 200
    idx, cb_pad = pl.pallas_call(
        _idx_body,
        out_shape=(jax.ShapeDtypeStruct((M,), jnp.int32),
                   jax.ShapeDtypeStruct((K, Dp), jnp.float32)),
    )(x_flat, cb_flat)
    out_p = _make_sc_gather(K, D, Dp, M)(idx, cb_pad)
    return out_p[:, :D].reshape(N, C, H, W)


# speedup vs baseline: 2.5529x; 2.5529x over previous
import jax
import jax.numpy as jnp
from jax.experimental import pallas as pl


def _body(x_ref, cb_ref, out_ref):
    M, D = x_ref.shape
    K = cb_ref.shape[0]
    xv = x_ref[...]
    cb = cb_ref[...]
    x2 = jnp.sum(xv * xv, axis=1, keepdims=True)
    c2 = jnp.sum(cb * cb, axis=1)[None, :]
    xc = jax.lax.dot_general(xv, cb, (((1,), (1,)), ((), ())),
                             preferred_element_type=jnp.float32)
    dist = x2 + c2 - 2.0 * xc
    mins = jnp.min(dist, axis=1, keepdims=True)
    kio = jax.lax.broadcasted_iota(jnp.int32, (M, K), 1).astype(jnp.float32)
    idx = jnp.min(jnp.where(dist == mins, kio, jnp.float32(K)),
                  axis=1, keepdims=True)
    onehot = jnp.where(kio == idx, jnp.float32(1), jnp.float32(0))
    out_ref[...] = jax.lax.dot_general(
        onehot, cb, (((1,), (0,)), ((), ())),
        preferred_element_type=jnp.float32)


def kernel(x, codebook):
    N, C, H, W = x.shape
    K = codebook.shape[0]
    D = H * W
    M = N * C
    x_flat = x.reshape(M, D)
    cb_flat = codebook.reshape(K, D)
    out = pl.pallas_call(
        _body,
        out_shape=jax.ShapeDtypeStruct((M, D), jnp.float32),
    )(x_flat, cb_flat)
    return out.reshape(N, C, H, W)
